# baseline (speedup 1.0000x reference)
import functools
import jax
import jax.numpy as jnp
from jax import lax
from jax.experimental import pallas as pl
from jax.experimental.pallas import tpu as pltpu
from jax.experimental.pallas import tpu_sc as plsc

VOCAB = 1000000
EMBED_DIM = 64
BATCH = 16384

_info = plsc.get_sparse_core_info()
_NC, _NS = _info.num_cores, _info.num_subcores
_W = 256
_CH = _W * _NS
_NCH = 122
_SCROWS = _CH * _NCH
_XBASE = _NC * _SCROWS
_XBLK = _NCH * _W
_TAILB = _XBLK + 2 * _W
_RING = 16
_ICH = 4096
_POSB = 14
_SENT = 1 << 20

_mesh = plsc.VectorSubcoreMesh(core_axis_name="c", subcore_axis_name="s")


@functools.partial(
    pl.kernel,
    mesh=_mesh,
    out_type=jax.ShapeDtypeStruct((BATCH, EMBED_DIM), jnp.float32),
    scratch_types=[
        pltpu.VMEM((_ICH,), jnp.int32),
        pltpu.VMEM((BATCH + 16,), jnp.int32),
        pltpu.VMEM((2, EMBED_DIM, _W), jnp.float32),
        pltpu.VMEM((16,), jnp.int32),
        pltpu.VMEM((_RING, EMBED_DIM), jnp.float32),
        pltpu.VMEM((EMBED_DIM, 64), jnp.float32),
        pltpu.VMEM_SHARED((3, EMBED_DIM, _CH), jnp.float32),
        pltpu.SemaphoreType.DMA,
        pltpu.SemaphoreType.DMA,
        pltpu.SemaphoreType.DMA,
        pltpu.SemaphoreType.DMA,
        pltpu.SemaphoreType.DMA,
        pltpu.SemaphoreType.DMA,
    ],
    compiler_params=pltpu.CompilerParams(needs_layout_passes=False),
)
def _gather(idx_hbm, tq_hbm, out_hbm, ichunk_v, lpk_v, slab_v, mpk_v,
            oring_v, tail_v, shared_v, sem_s0, sem_s1, sem_c0, sem_c1,
            sem_c2, sem_out):
    c = lax.axis_index("c")
    s = lax.axis_index("s")
    scbase = c * _SCROWS
    is_x = jnp.logical_and(c == _NC - 1, s == _NS - 1)

    _SEM_SL = (sem_s0, sem_s1)
    _SEM_CH = (sem_c0, sem_c1, sem_c2)

    def issue_chunk(k):
        base = scbase + k * _CH
        for cs in range(3):
            @pl.when(lax.rem(k, 3) == cs)
            def _(cs=cs):
                for tc in range(EMBED_DIM // 8):
                    pltpu.async_copy(
                        tq_hbm.at[pl.ds(tc * 8, 8), pl.ds(base, _CH)],
                        shared_v.at[cs, pl.ds(tc * 8, 8)],
                        _SEM_CH[cs],
                    )

    def drain_chunk(k):
        for cs in range(3):
            @pl.when(lax.rem(k, 3) == cs)
            def _(cs=cs):
                pltpu.make_async_copy(
                    tq_hbm.at[pl.ds(0, EMBED_DIM), pl.ds(0, _CH)],
                    shared_v.at[cs],
                    _SEM_CH[cs],
                ).wait()

    def issue_slab(k):
        cs3 = lax.rem(k, 3)
        off = pl.multiple_of(s * _W, _W)
        for sl in range(2):
            @pl.when(lax.rem(k, 2) == sl)
            def _(sl=sl):
                pltpu.async_copy(
                    shared_v.at[cs3, pl.ds(0, EMBED_DIM), pl.ds(off, _W)],
                    slab_v.at[sl],
                    _SEM_SL[sl],
                )

    def drain_slab(k):
        for sl in range(2):
            @pl.when(lax.rem(k, 2) == sl)
            def _(sl=sl):
                pltpu.make_async_copy(
                    tq_hbm.at[pl.ds(0, EMBED_DIM), pl.ds(0, _W)],
                    slab_v.at[sl],
                    _SEM_SL[sl],
                ).wait()

    @pl.when(s == 0)
    def _():
        issue_chunk(jnp.int32(0))
        issue_chunk(jnp.int32(1))

    def scan_chunk(ci, n):
        pltpu.sync_copy(idx_hbm.at[pl.ds(ci * _ICH, _ICH)], ichunk_v)

        def scan_g(g, n):
            v = ichunk_v[pl.ds(g * 16, 16)]
            p = lax.iota(jnp.int32, 16) + (ci * _ICH + g * 16)
            rel = v - scbase
            in_main = jnp.logical_and(v >= scbase, v < scbase + _SCROWS)
            stripe = lax.rem(lax.shift_right_logical(rel, 8), _NS)
            m_main = jnp.logical_and(in_main, stripe == s)
            m_x = jnp.logical_and(v >= _XBASE, is_x)
            m = jnp.logical_or(m_main, m_x)
            rt_main = lax.shift_left(lax.shift_right_logical(rel, 12), 8) | (
                rel & (_W - 1)
            )
            rt_x = _XBLK + (v - _XBASE)
            rt = jnp.where(m_x, rt_x, rt_main)
            pk = lax.shift_left(rt, _POSB) | p
            plsc.store_compressed(lpk_v.at[pl.ds(n, 16)], pk, mask=m)
            return n + plsc.all_reduce_population_count(m)[0]

        return lax.fori_loop(0, _ICH // 16, scan_g, n)

    n = lax.fori_loop(0, BATCH // _ICH, scan_chunk, jnp.int32(0))
    ng = (n + 15) // 16

    def scan_and_emit(r0rel, gather_row, issued):
        def scan_list(g, issued):
            q = lpk_v[pl.ds(g * 16, 16)]
            rel = lax.shift_right_logical(q, _POSB)
            m = jnp.logical_and(rel >= r0rel, rel < r0rel + _W)
            plsc.store_compressed(mpk_v.at[pl.ds(0, 16)], q, mask=m)
            cnt = plsc.all_reduce_population_count(m)[0]

            def emit(j, issued):
                jv = jnp.full((16,), 0, jnp.int32) + j
                q1 = plsc.load_gather(mpk_v.at[pl.ds(0, 16)], [jv])[0]
                pos = q1 & ((1 << _POSB) - 1)
                relv = (
                    jnp.full((16,), 0, jnp.int32)
                    + (lax.shift_right_logical(q1, _POSB) - r0rel)
                )
                oslot = lax.rem(issued, _RING)

                @pl.when(issued >= _RING)
                def _():
                    pltpu.make_async_copy(
                        out_hbm.at[0], oring_v.at[0], sem_out
                    ).wait()

                for k in range(EMBED_DIM // 16):
                    cvec = lax.iota(jnp.int32, 16) + 16 * k
                    oring_v[oslot, pl.ds(16 * k, 16)] = gather_row(cvec, relv)
                pltpu.async_copy(oring_v.at[oslot], out_hbm.at[pos], sem_out)
                return issued + 1

            return lax.fori_loop(0, cnt, emit, issued)

        return lax.fori_loop(0, ng, scan_list, issued)

    def gather_slab(slotv):
        def gather_row(cvec, relv):
            return plsc.load_gather(
                slab_v.at[pl.ds(0, 2), pl.ds(0, EMBED_DIM), pl.ds(0, _W)],
                [slotv, cvec, relv],
            )

        return gather_row

    def do_round(k, issued):
        @pl.when(k >= 1)
        def _():
            drain_slab(k - 1)

        plsc.subcore_barrier()

        @pl.when(s == 0)
        def _():
            @pl.when(k + 2 < _NCH)
            def _():
                issue_chunk(k + 2)

            @pl.when(k < _NCH)
            def _():
                drain_chunk(k)

        plsc.subcore_barrier()

        @pl.when(k < _NCH)
        def _():
            issue_slab(k)

        r0rel = jnp.where(k >= 1, (k - 1) * _W, _SENT)
        slotv = jnp.full((16,), 0, jnp.int32) + lax.rem(k + 1, 2)
        return scan_and_emit(r0rel, gather_slab(slotv), issued)

    issued = lax.fori_loop(0, _NCH + 1, do_round, jnp.int32(0))

    @pl.when(is_x)
    def _():
        for tc in range(EMBED_DIM // 8):
            pltpu.sync_copy(
                tq_hbm.at[pl.ds(tc * 8, 8), pl.ds(_XBASE, _W)],
                slab_v.at[0, pl.ds(tc * 8, 8)],
            )
            pltpu.sync_copy(
                tq_hbm.at[pl.ds(tc * 8, 8), pl.ds(_XBASE + _W, _W)],
                slab_v.at[1, pl.ds(tc * 8, 8)],
            )
            pltpu.sync_copy(
                tq_hbm.at[pl.ds(tc * 8, 8), pl.ds(_XBASE + 2 * _W, 64)],
                tail_v.at[pl.ds(tc * 8, 8)],
            )

    issued = scan_and_emit(
        jnp.int32(_XBLK),
        gather_slab(jnp.full((16,), 0, jnp.int32)),
        issued,
    )
    issued = scan_and_emit(
        jnp.int32(_XBLK + _W),
        gather_slab(jnp.full((16,), 1, jnp.int32)),
        issued,
    )

    def gather_tail(cvec, relv):
        return plsc.load_gather(
            tail_v.at[pl.ds(0, EMBED_DIM), pl.ds(0, 64)], [cvec, relv]
        )

    issued = scan_and_emit(jnp.int32(_TAILB), gather_tail, issued)

    def drain_out(i, _):
        pltpu.make_async_copy(out_hbm.at[0], oring_v.at[0], sem_out).wait()
        return 0

    lax.fori_loop(0, jnp.minimum(issued, _RING), drain_out, 0)


def kernel(x, table):
    out = _gather(x.reshape(BATCH), table.T)
    return out.reshape(BATCH, 1, EMBED_DIM)
